# Initial kernel scaffold; baseline (speedup 1.0000x reference)
import jax
import jax.numpy as jnp
from jax.experimental import pallas as pl


def kernel(x, edge_index, edge_attr, p_w, p_b, q_w, q_b):
    raise NotImplementedError("write your pallas kernel here")


# speedup vs baseline: 119.1043x; 119.1043x over previous
import jax
import jax.numpy as jnp
from jax import lax
from jax.experimental import pallas as pl
from jax.experimental.pallas import tpu as pltpu
from jax.experimental.pallas import tpu_sc as plsc

N = 10000
E = 320000
D = 128
NP = 10240
EP = 327680
EM = EP // 32
HC = EM // 128
NCH = EP // 128

_MESH = plsc.VectorSubcoreMesh(core_axis_name="c", subcore_axis_name="s")
_SC_PARAMS = pltpu.CompilerParams(needs_layout_passes=False)


def _hist_body(col2_hbm, zero_hbm, deg_hbm, colv, onesv, deg_sh):
    c = lax.axis_index("c")
    s = lax.axis_index("s")
    wid = c * 16 + s

    @pl.when(s == 0)
    def _():
        pltpu.sync_copy(zero_hbm, deg_sh)

    pltpu.sync_copy(col2_hbm.at[pl.ds(wid * HC, HC)], colv)
    for j in range(8):
        onesv[pl.ds(j * 16, 16)] = jnp.full((16,), 1.0, dtype=jnp.float32)
    plsc.subcore_barrier()

    def chunk(j, carry):
        pltpu.sync_copy(onesv, deg_sh.at[colv.at[j]], add=True)
        return carry

    lax.fori_loop(0, HC, chunk, 0)
    plsc.subcore_barrier()

    @pl.when(s == 0)
    def _():
        pltpu.sync_copy(deg_sh, deg_hbm.at[c])


_hist = pl.kernel(
    _hist_body,
    out_type=jax.ShapeDtypeStruct((2, NP), jnp.float32),
    mesh=_MESH,
    compiler_params=_SC_PARAMS,
    scratch_types=[
        pltpu.VMEM((HC, 128), jnp.int32),
        pltpu.VMEM((128,), jnp.float32),
        pltpu.VMEM_SHARED((NP,), jnp.float32),
    ],
)


def _map_body(row_hbm, col_hbm, ea_hbm, deg_hbm, pv_hbm, qv_hbm, out_hbm,
              d0v, d1v, pvv, qvv, uv, pcv, rv, cv, eav, ov):
    c = lax.axis_index("c")
    s = lax.axis_index("s")
    wid = c * 16 + s
    base = wid * EM

    pltpu.sync_copy(deg_hbm.at[0], d0v)
    pltpu.sync_copy(deg_hbm.at[1], d1v)
    pltpu.sync_copy(pv_hbm, pvv)
    pltpu.sync_copy(qv_hbm, qvv)
    pltpu.sync_copy(row_hbm.at[pl.ds(base, EM)], rv)
    pltpu.sync_copy(col_hbm.at[pl.ds(base, EM)], cv)
    pltpu.sync_copy(ea_hbm.at[pl.ds(base, EM)], eav)

    def table(i, carry):
        sl = pl.ds(i * 16, 16)
        d = d0v[sl] + d1v[sl]
        half = d * 0.5
        y = lax.bitcast_convert_type(
            jnp.int32(0x5F3759DF) - (lax.bitcast_convert_type(d, jnp.int32) >> 1),
            jnp.float32)
        y = y * (1.5 - half * y * y)
        y = y * (1.5 - half * y * y)
        y = y * (1.5 - half * y * y)
        u = jnp.where(d == 0.0, jnp.full((16,), jnp.inf, jnp.float32), y)
        uv[sl] = u
        pcv[sl] = u * pvv[sl]
        return carry

    lax.fori_loop(0, NP // 16, table, 0)

    def edge(i, carry):
        sl = pl.ds(i * 16, 16)
        ir = rv[sl]
        ic = cv[sl]
        ur = plsc.load_gather(uv, [ir])
        qr = plsc.load_gather(qvv, [ir])
        pcc = plsc.load_gather(pcv, [ic])
        ov[sl] = eav[sl] * (ur * pcc + qr)
        return carry

    lax.fori_loop(0, EM // 16, edge, 0)
    pltpu.sync_copy(ov, out_hbm.at[pl.ds(base, EM)])


_map = pl.kernel(
    _map_body,
    out_type=jax.ShapeDtypeStruct((EP,), jnp.float32),
    mesh=_MESH,
    compiler_params=_SC_PARAMS,
    scratch_types=[
        pltpu.VMEM((NP,), jnp.float32),
        pltpu.VMEM((NP,), jnp.float32),
        pltpu.VMEM((NP,), jnp.float32),
        pltpu.VMEM((NP,), jnp.float32),
        pltpu.VMEM((NP,), jnp.float32),
        pltpu.VMEM((NP,), jnp.float32),
        pltpu.VMEM((EM,), jnp.int32),
        pltpu.VMEM((EM,), jnp.int32),
        pltpu.VMEM((EM,), jnp.float32),
        pltpu.VMEM((EM,), jnp.float32),
    ],
)


def _mv_body(x_ref, w_ref, b_ref, o_ref):
    o_ref[...] = jnp.maximum(
        jnp.dot(x_ref[...], w_ref[...], preferred_element_type=jnp.float32)
        + b_ref[...], 0.0)


def _matvec(x, w, b):
    return pl.pallas_call(
        _mv_body,
        out_shape=jax.ShapeDtypeStruct((N, 2), jnp.float32),
    )(x, w, b)


def kernel(x, edge_index, edge_attr, p_w, p_b, q_w, q_b):
    ei = edge_index.astype(jnp.int32)
    row = jnp.concatenate([ei[0], jnp.zeros((EP - E,), jnp.int32)])
    col = jnp.concatenate([ei[1], jnp.full((EP - E,), N, jnp.int32)])
    ea = jnp.concatenate([edge_attr, jnp.zeros((EP - E,), jnp.float32)])
    col2 = col.reshape(NCH, 128)
    zero = jnp.zeros((NP,), jnp.float32)

    w = jnp.concatenate([p_w, q_w], axis=0).T
    b = jnp.concatenate([p_b, q_b]).reshape(1, 2)
    pq = _matvec(x, w, b)
    padn = jnp.zeros((NP - N,), jnp.float32)
    pv = jnp.concatenate([pq[:, 0], padn])
    qv = jnp.concatenate([pq[:, 1], padn])

    deg = _hist(col2, zero)
    out = _map(row, col, ea, deg, pv, qv)
    return (edge_index, out[:E])


# speedup vs baseline: 145.3755x; 1.2206x over previous
import jax
import jax.numpy as jnp
from jax import lax
from jax.experimental import pallas as pl
from jax.experimental.pallas import tpu as pltpu
from jax.experimental.pallas import tpu_sc as plsc

N = 10000
E = 320000
D = 128
NS = 10112
HC = 80
EP = 32 * HC * 128
EM = E // 32

_MESH = plsc.VectorSubcoreMesh(core_axis_name="c", subcore_axis_name="s")
_SC_PARAMS = pltpu.CompilerParams(needs_layout_passes=False)


def _hist_body(col2_hbm, zero_hbm, ones_hbm, deg_hbm, colv, onesv, deg_sh, sem):
    c = lax.axis_index("c")
    s = lax.axis_index("s")
    wid = c * 16 + s

    d1 = pltpu.async_copy(col2_hbm.at[pl.ds(wid * HC, HC)], colv, sem)
    d2 = pltpu.async_copy(ones_hbm, onesv, sem)

    @pl.when(s == 0)
    def _():
        pltpu.sync_copy(zero_hbm, deg_sh)

    d1.wait()
    d2.wait()
    plsc.subcore_barrier()
    descs = [pltpu.async_copy(onesv, deg_sh.at[colv.at[j]], add=True, sem=sem)
             for j in range(HC)]
    for d in descs:
        d.wait()
    plsc.subcore_barrier()

    @pl.when(s == 0)
    def _():
        pltpu.sync_copy(deg_sh, deg_hbm.at[pl.ds(c * NS, NS)])


_hist = pl.kernel(
    _hist_body,
    out_type=jax.ShapeDtypeStruct((2 * NS,), jnp.float32),
    mesh=_MESH,
    compiler_params=_SC_PARAMS,
    scratch_types=[
        pltpu.VMEM((HC, 128), jnp.int32),
        pltpu.VMEM((128,), jnp.float32),
        pltpu.VMEM_SHARED((NS,), jnp.float32),
        pltpu.SemaphoreType.DMA,
    ],
)


def _map_body(ei_hbm, ea_hbm, deg_hbm, pq_hbm, out_hbm,
              d0v, d1v, pvv, qvv, uv, pcv, rv, cv, eav, ov, semA, semB):
    c = lax.axis_index("c")
    s = lax.axis_index("s")
    wid = c * 16 + s
    base = wid * EM

    a1 = pltpu.async_copy(deg_hbm.at[pl.ds(0, N)], d0v, semA)
    a2 = pltpu.async_copy(deg_hbm.at[pl.ds(NS, N)], d1v, semA)
    a3 = pltpu.async_copy(pq_hbm.at[pl.ds(0, N)], pvv, semA)
    b1 = pltpu.async_copy(ei_hbm.at[pl.ds(base, EM)], rv, semB)
    b2 = pltpu.async_copy(ei_hbm.at[pl.ds(E + base, EM)], cv, semB)
    b3 = pltpu.async_copy(ea_hbm.at[pl.ds(base, EM)], eav, semB)
    b4 = pltpu.async_copy(pq_hbm.at[pl.ds(N, N)], qvv, semB)
    a1.wait()
    a2.wait()
    a3.wait()

    @plsc.parallel_loop(0, N, step=16, unroll=4)
    def _table(i):
        sl = pl.ds(i, 16)
        d = d0v[sl] + d1v[sl]
        half = d * 0.5
        y = lax.bitcast_convert_type(
            jnp.int32(0x5F3759DF) - (lax.bitcast_convert_type(d, jnp.int32) >> 1),
            jnp.float32)
        y = y * (1.5 - half * y * y)
        y = y * (1.5 - half * y * y)
        y = y * (1.5 - half * y * y)
        u = jnp.where(d == 0.0, jnp.full((16,), jnp.inf, jnp.float32), y)
        uv[sl] = u
        pcv[sl] = u * pvv[sl]

    b1.wait()
    b2.wait()
    b3.wait()
    b4.wait()

    @plsc.parallel_loop(0, EM, step=16, unroll=4)
    def _edge(i):
        sl = pl.ds(i, 16)
        ir = rv[sl]
        ic = cv[sl]
        ur = plsc.load_gather(uv, [ir])
        qr = plsc.load_gather(qvv, [ir])
        pcc = plsc.load_gather(pcv, [ic])
        ov[sl] = eav[sl] * (ur * pcc + qr)

    pltpu.sync_copy(ov, out_hbm.at[pl.ds(base, EM)])


_map = pl.kernel(
    _map_body,
    out_type=jax.ShapeDtypeStruct((E,), jnp.float32),
    mesh=_MESH,
    compiler_params=_SC_PARAMS,
    scratch_types=[
        pltpu.VMEM((N,), jnp.float32),
        pltpu.VMEM((N,), jnp.float32),
        pltpu.VMEM((N,), jnp.float32),
        pltpu.VMEM((N,), jnp.float32),
        pltpu.VMEM((N,), jnp.float32),
        pltpu.VMEM((N,), jnp.float32),
        pltpu.VMEM((EM,), jnp.int32),
        pltpu.VMEM((EM,), jnp.int32),
        pltpu.VMEM((EM,), jnp.float32),
        pltpu.VMEM((EM,), jnp.float32),
        pltpu.SemaphoreType.DMA,
        pltpu.SemaphoreType.DMA,
    ],
)


def _mv_body(x_ref, w_ref, b_ref, o_ref):
    o_ref[...] = jnp.maximum(
        lax.dot_general(w_ref[...], x_ref[...],
                        (((1,), (1,)), ((), ())),
                        preferred_element_type=jnp.float32)
        + b_ref[...], 0.0)


def _matvec(x, w, b):
    return pl.pallas_call(
        _mv_body,
        out_shape=jax.ShapeDtypeStruct((2, N), jnp.float32),
    )(x, w, b)


def kernel(x, edge_index, edge_attr, p_w, p_b, q_w, q_b):
    ei = edge_index.astype(jnp.int32)
    col2 = jnp.concatenate(
        [ei[1], jnp.full((EP - E,), NS - 1, jnp.int32)]).reshape(EP // 128, 128)
    zero = jnp.zeros((NS,), jnp.float32)
    ones = jnp.ones((128,), jnp.float32)

    w = jnp.concatenate([p_w, q_w], axis=0)
    b = jnp.concatenate([p_b, q_b]).reshape(2, 1)
    pq = _matvec(x, w, b).reshape(-1)

    deg = _hist(col2, zero, ones)
    out = _map(ei.reshape(-1), edge_attr, deg, pq)
    return (edge_index, out)
